# Initial kernel scaffold; baseline (speedup 1.0000x reference)
import functools

import jax
import jax.numpy as jnp
from jax import lax
from jax.experimental import pallas as pl
from jax.experimental.pallas import tpu as pltpu
from jax.experimental.pallas import tpu_sc as plsc

BATCH = 64
SEQ = 50
RANK = 64
VOCAB = 1000

NC = 2
NS = 16
NW = NC * NS
ROWS_PER_W = BATCH // NW
CHUNK = 8
NCHUNK = (SEQ + CHUNK - 1) // CHUNK
NBLK = RANK // 16


_VB = 40


def _sq_transpose_body(c_ref, t_ref):
    for k in range(_VB):
        blk = c_ref[:, k, :]
        t_ref[k] = blk * blk


def _build_table(core3):
    return pl.pallas_call(
        _sq_transpose_body,
        grid=(VOCAB // _VB,),
        in_specs=[pl.BlockSpec((RANK, _VB, RANK), lambda i: (0, i, 0))],
        out_specs=pl.BlockSpec((_VB, RANK, RANK), lambda i: (i, 0, 0)),
        out_shape=jax.ShapeDtypeStruct((VOCAB, RANK, RANK), jnp.float32),
    )(core3)


def _sc_body(table, y, alpha, beta, out, y_v, a_v, b_v, carry, buf, out_v, sem):
    wid = lax.axis_index("s") * NC + lax.axis_index("c")
    pltpu.sync_copy(alpha, a_v)
    pltpu.sync_copy(beta, b_v)
    zeros_i = jnp.zeros((16,), jnp.int32)

    for row_i in range(ROWS_PER_W):
        b = wid * ROWS_PER_W + row_i
        y_v[pl.ds(48, 16)] = zeros_i
        pltpu.sync_copy(y.at[b], y_v.at[pl.ds(0, SEQ)])

        for blk in range(NBLK):
            a = a_v[pl.ds(blk * 16, 16)]
            carry[pl.ds(blk * 16, 16)] = a * a

        def chunk_body(c, _):
            idx = y_v.at[pl.ds(c * CHUNK, CHUNK)]
            pltpu.async_copy(table.at[idx], buf, sem).wait()
            nsteps = jnp.where(c == NCHUNK - 1, SEQ - (NCHUNK - 1) * CHUNK,
                               CHUNK)

            def step(j, _):
                acc = [jnp.zeros((16,), jnp.float32) for _ in range(NBLK)]
                for r in range(RANK):
                    cb = plsc.load_gather(
                        carry, [jnp.full((16,), r, jnp.int32)])
                    for blk in range(NBLK):
                        m = buf[j, r, pl.ds(blk * 16, 16)]
                        acc[blk] = acc[blk] + cb * m
                for blk in range(NBLK):
                    carry[pl.ds(blk * 16, 16)] = acc[blk]
                return 0

            lax.fori_loop(0, nsteps, step, 0)
            return 0

        lax.fori_loop(0, NCHUNK, chunk_body, 0)

        tot = jnp.zeros((16,), jnp.float32)
        for blk in range(NBLK):
            bb = b_v[pl.ds(blk * 16, 16)]
            tot = tot + carry[pl.ds(blk * 16, 16)] * bb * bb
        out_v[row_i] = jnp.sum(tot)

    pltpu.sync_copy(out_v, out.at[wid])


@functools.partial(
    pl.kernel,
    out_type=jax.ShapeDtypeStruct((NW, 8), jnp.float32),
    mesh=plsc.VectorSubcoreMesh(
        core_axis_name="c", subcore_axis_name="s", num_cores=NC,
        num_subcores=NS),
    scratch_types=[
        pltpu.VMEM((64,), jnp.int32),
        pltpu.VMEM((RANK,), jnp.float32),
        pltpu.VMEM((RANK,), jnp.float32),
        pltpu.VMEM((RANK,), jnp.float32),
        pltpu.VMEM((CHUNK, RANK, RANK), jnp.float32),
        pltpu.VMEM((8,), jnp.float32),
        pltpu.SemaphoreType.DMA,
    ],
)
def _sc_chain(table, y, alpha, beta, out, *scratch):
    _sc_body(table, y, alpha, beta, out, *scratch)


def kernel(y, alpha, beta, core):
    table = _build_table(core[0])
    out2d = _sc_chain(table, y.astype(jnp.int32), alpha[0], beta[0])
    return out2d[:, :ROWS_PER_W].reshape(BATCH)


# speedup vs baseline: 2.3920x; 2.3920x over previous
import functools

import jax
import jax.numpy as jnp
from jax import lax
from jax.experimental import pallas as pl
from jax.experimental.pallas import tpu as pltpu
from jax.experimental.pallas import tpu_sc as plsc

BATCH = 64
SEQ = 50
RANK = 64
VOCAB = 1000

NC = 2
NS = 16
NW = NC * NS
ROWS_PER_W = BATCH // NW
CHUNK = 8
NCHUNK = (SEQ + CHUNK - 1) // CHUNK
SEQ_PAD = NCHUNK * CHUNK
NBLK = RANK // 16


_VB = 40


def _sq_transpose_body(c_ref, t_ref):
    for k in range(_VB):
        blk = c_ref[:, k, :]
        t_ref[k] = blk * blk


def _build_table(core3):
    return pl.pallas_call(
        _sq_transpose_body,
        grid=(VOCAB // _VB,),
        in_specs=[pl.BlockSpec((RANK, _VB, RANK), lambda i: (0, i, 0))],
        out_specs=pl.BlockSpec((_VB, RANK, RANK), lambda i: (i, 0, 0)),
        out_shape=jax.ShapeDtypeStruct((VOCAB, RANK, RANK), jnp.float32),
    )(core3)


def _sc_body(table, y, alpha, beta, out, y_v, idx_v, a_v, b_v, buf, out_v,
             sem):
    wid = lax.axis_index("s") * NC + lax.axis_index("c")
    pltpu.sync_copy(alpha, a_v)
    pltpu.sync_copy(beta, b_v)
    lanes = lax.broadcasted_iota(jnp.int32, (16,), 0)
    out_vec = jnp.zeros((16,), jnp.float32)

    for row_i in range(ROWS_PER_W):
        b = wid * ROWS_PER_W + row_i
        pltpu.sync_copy(y.at[pl.ds(b * SEQ_PAD, SEQ_PAD)],
                        y_v.at[pl.ds(0, SEQ_PAD)])

        cr0 = []
        for blk in range(NBLK):
            a = a_v[pl.ds(blk * 16, 16)]
            cr0.append(a * a)

        def chunk_body(c, cr):
            idx_v[...] = y_v[pl.ds(c * CHUNK, 16)]
            pltpu.async_copy(table.at[idx_v.at[pl.ds(0, CHUNK)]], buf,
                             sem).wait()
            nsteps = jnp.where(c == NCHUNK - 1, SEQ - (NCHUNK - 1) * CHUNK,
                               CHUNK)

            def step(j, cr):
                acc = [jnp.zeros((16,), jnp.float32) for _ in range(NBLK)]
                for rb in range(NBLK):
                    cvec = cr[rb]
                    for rl in range(16):
                        r = rb * 16 + rl
                        cs = jnp.sum(jnp.where(lanes == rl, cvec, 0.0))
                        cb = jnp.full((16,), cs)
                        for blk in range(NBLK):
                            m = buf[j, pl.ds(r * RANK + blk * 16, 16)]
                            acc[blk] = acc[blk] + cb * m
                return tuple(acc)

            return lax.fori_loop(0, nsteps, step, cr)

        cr = lax.fori_loop(0, NCHUNK, chunk_body, tuple(cr0))

        tot = jnp.zeros((16,), jnp.float32)
        for blk in range(NBLK):
            bb = b_v[pl.ds(blk * 16, 16)]
            tot = tot + cr[blk] * bb * bb
        total = jnp.sum(tot)
        out_vec = jnp.where(lanes == row_i, jnp.full((16,), total), out_vec)

    out_v[...] = out_vec
    pltpu.sync_copy(out_v, out.at[pl.ds(wid * 16, 16)])


@functools.partial(
    pl.kernel,
    out_type=jax.ShapeDtypeStruct((NW * 16,), jnp.float32),
    mesh=plsc.VectorSubcoreMesh(
        core_axis_name="c", subcore_axis_name="s", num_cores=NC,
        num_subcores=NS),
    compiler_params=pltpu.CompilerParams(needs_layout_passes=False),
    scratch_types=[
        pltpu.VMEM((64,), jnp.int32),
        pltpu.VMEM((16,), jnp.int32),
        pltpu.VMEM((RANK,), jnp.float32),
        pltpu.VMEM((RANK,), jnp.float32),
        pltpu.VMEM((CHUNK, RANK * RANK), jnp.float32),
        pltpu.VMEM((16,), jnp.float32),
        pltpu.SemaphoreType.DMA,
    ],
)
def _sc_chain(table, y, alpha, beta, out, *scratch):
    _sc_body(table, y, alpha, beta, out, *scratch)


def kernel(y, alpha, beta, core):
    table = _build_table(core[0]).reshape(VOCAB, RANK * RANK)
    y_pad = jnp.pad(y.astype(jnp.int32), ((0, 0), (0, SEQ_PAD - SEQ)))
    out_flat = _sc_chain(table, y_pad.reshape(-1), alpha[0], beta[0])
    return out_flat.reshape(NW, 16)[:, :ROWS_PER_W].reshape(BATCH)
